# baseline (speedup 1.0000x reference)
import functools

import jax
import jax.numpy as jnp
from jax import lax
from jax.experimental import pallas as pl
from jax.experimental.pallas import tpu as pltpu
from jax.experimental.pallas import tpu_sc as plsc

B, S_MAX, T_PROMPT, L_TARGET, D, V = 8, 512, 256, 256, 2048, 32000
AUDIO_TOKEN_ID = 5
PAD_TOKEN_ID = 0
MAX_LEN = S_MAX + T_PROMPT + L_TARGET

P_SLOTS = 32
E_SLOTS = 384
A_SLOTS = 512
Z_SLOTS = 640
E_CH, A_CH, Z_CH = 32, 32, 16
E_CHUNKS = B * E_SLOTS // (32 * E_CH)
A_CHUNKS = B * A_SLOTS // (32 * A_CH)
Z_CHUNKS = B * Z_SLOTS // (32 * Z_CH)


def _sc_pack(embed_table, proj_flat, zeros_src, eidx, edst, aidx, adst, zdst):
  info = plsc.get_sparse_core_info()
  NC, NS = info.num_cores, info.num_subcores
  mesh = plsc.VectorSubcoreMesh(core_axis_name="c", subcore_axis_name="s")

  @functools.partial(
      pl.kernel, mesh=mesh,
      out_type=jax.ShapeDtypeStruct((B * MAX_LEN, D), jnp.float32),
      scratch_types=[
          pltpu.VMEM((E_CHUNKS, E_CH), jnp.int32),
          pltpu.VMEM((E_CHUNKS, E_CH), jnp.int32),
          pltpu.VMEM((A_CHUNKS, A_CH), jnp.int32),
          pltpu.VMEM((A_CHUNKS, A_CH), jnp.int32),
          pltpu.VMEM((Z_CHUNKS, Z_CH), jnp.int32),
          pltpu.VMEM((E_CH, D), jnp.float32),
          pltpu.VMEM((Z_CH, D), jnp.float32),
          pltpu.SemaphoreType.DMA,
      ],
  )
  def k(table_hbm, proj_hbm, zsrc_hbm, eidx_hbm, edst_hbm, aidx_hbm,
        adst_hbm, zdst_hbm, out_hbm, eidx_v, edst_v, aidx_v, adst_v,
        zdst_v, rows_v, zbuf_v, sem):
    wid = lax.axis_index("s") * NC + lax.axis_index("c")
    pltpu.sync_copy(eidx_hbm.at[wid], eidx_v)
    pltpu.sync_copy(edst_hbm.at[wid], edst_v)
    pltpu.sync_copy(aidx_hbm.at[wid], aidx_v)
    pltpu.sync_copy(adst_hbm.at[wid], adst_v)
    pltpu.sync_copy(zdst_hbm.at[wid], zdst_v)
    pltpu.sync_copy(zsrc_hbm, zbuf_v)
    for j in range(E_CHUNKS):
      pltpu.async_copy(table_hbm.at[eidx_v.at[j]], rows_v, sem).wait()
      pltpu.async_copy(rows_v, out_hbm.at[edst_v.at[j]], sem).wait()
    for j in range(A_CHUNKS):
      pltpu.async_copy(proj_hbm.at[aidx_v.at[j]], rows_v, sem).wait()
      pltpu.async_copy(rows_v, out_hbm.at[adst_v.at[j]], sem).wait()
    for j in range(Z_CHUNKS):
      pltpu.async_copy(zbuf_v, out_hbm.at[zdst_v.at[j]], sem).wait()

  return k(embed_table, proj_flat, zeros_src, eidx, edst, aidx, adst, zdst)


def _text_norm_kernel(ids_ref, row_ref, out_ref, acc_ref):
  i = pl.program_id(0)
  n = pl.num_programs(0)

  @pl.when(i == 0)
  def _():
    acc_ref[0] = 0.0
    acc_ref[1] = 0.0

  row = row_ref[0]
  nrm = jnp.sqrt(jnp.sum(row * row))
  valid = (ids_ref[i] != PAD_TOKEN_ID).astype(jnp.float32)
  acc_ref[0] = acc_ref[0] + nrm * valid
  acc_ref[1] = acc_ref[1] + valid

  @pl.when(i == n - 1)
  def _():
    out_ref[...] = (acc_ref[0] / jnp.maximum(acc_ref[1], 1.0)).reshape(1, 1)


def _text_norm(prompt_ids, embed_table):
  ids = prompt_ids.reshape(-1).astype(jnp.int32)
  grid_spec = pltpu.PrefetchScalarGridSpec(
      num_scalar_prefetch=1,
      grid=(B * T_PROMPT,),
      in_specs=[pl.BlockSpec((1, 1, D), lambda i, ids: (ids[i], 0, 0))],
      out_specs=pl.BlockSpec((1, 1), lambda i, ids: (0, 0)),
      scratch_shapes=[pltpu.SMEM((2,), jnp.float32)],
  )
  out = pl.pallas_call(
      _text_norm_kernel, grid_spec=grid_spec,
      out_shape=jax.ShapeDtypeStruct((1, 1), jnp.float32),
  )(ids, embed_table.reshape(V, 1, D))
  return out[0, 0]


def _audio_norm_kernel(proj_ref, alen_ref, out_ref, acc_ref):
  b = pl.program_id(0)

  @pl.when(b == 0)
  def _():
    acc_ref[0] = 0.0
    acc_ref[1] = 0.0

  x = proj_ref[0]
  sumsq = jnp.sum(x * x, axis=-1)
  nrm = jnp.sqrt(sumsq)
  mask = (lax.broadcasted_iota(jnp.int32, (S_MAX,), 0)
          < alen_ref[0, 0, 0]).astype(jnp.float32)
  acc_ref[0] = acc_ref[0] + jnp.sum(nrm * mask)
  acc_ref[1] = acc_ref[1] + jnp.sum(mask)

  @pl.when(b == B - 1)
  def _():
    out_ref[...] = (acc_ref[0] / jnp.maximum(acc_ref[1], 1.0)).reshape(1, 1)


def _audio_norm(proj_embs, audio_lens):
  out = pl.pallas_call(
      _audio_norm_kernel,
      grid=(B,),
      in_specs=[
          pl.BlockSpec((1, S_MAX, D), lambda b: (b, 0, 0)),
          pl.BlockSpec((1, 1, 1), lambda b: (b, 0, 0)),
      ],
      out_specs=pl.BlockSpec((1, 1), lambda b: (0, 0)),
      scratch_shapes=[pltpu.SMEM((2,), jnp.float32)],
      out_shape=jax.ShapeDtypeStruct((1, 1), jnp.float32),
  )(proj_embs, audio_lens.astype(jnp.int32).reshape(B, 1, 1))
  return out[0, 0]


def _meta_kernel(pids_ref, tids_ref, alen_ref, am_ref, lab_ref, pos_ref):
  pids = pids_ref[0, 0].astype(jnp.int32)
  tids = tids_ref[0, 0].astype(jnp.int32)
  alen = alen_ref[0, 0, 0]
  p = lax.broadcasted_iota(jnp.int32, (1, MAX_LEN), 1)
  tp = lax.broadcasted_iota(jnp.int32, (1, T_PROMPT), 1)
  audio_pos = jnp.min(jnp.where(pids == AUDIO_TOKEN_ID, tp[0], T_PROMPT))
  plen = jnp.sum((pids != PAD_TOKEN_ID).astype(jnp.int32))
  tlen = jnp.sum((tids != PAD_TOKEN_ID).astype(jnp.int32))
  tstart = alen + plen

  oh = (p.reshape(MAX_LEN, 1) ==
        (tstart + lax.broadcasted_iota(jnp.int32, (1, L_TARGET), 1)
         .reshape(1, L_TARGET))).astype(jnp.float32)
  placed = jax.lax.dot_general(
      oh, tids.astype(jnp.float32).reshape(L_TARGET, 1),
      (((1,), (0,)), ((), ())),
      preferred_element_type=jnp.float32).reshape(1, MAX_LEN)
  in_tgt_valid = (p >= tstart) & (p < tstart + tlen)
  lab = jnp.where(in_tgt_valid, placed.astype(jnp.int32), -100)
  lab = jnp.where((p == 0) & (tlen < L_TARGET), 0, lab)
  lab_ref[...] = lab.reshape(1, 1, MAX_LEN)

  in_tgt_win = (p >= tstart) & (p < tstart + L_TARGET)
  in_audio = (p >= audio_pos) & (p < audio_pos + alen)
  pna = (pids != PAD_TOKEN_ID) & (pids != AUDIO_TOKEN_ID)
  pna_full = jnp.concatenate(
      [pna.reshape(1, T_PROMPT),
       jnp.zeros((1, MAX_LEN - T_PROMPT), dtype=jnp.bool_)], axis=1)
  tmask_full = jnp.concatenate(
      [(tids != PAD_TOKEN_ID).reshape(1, L_TARGET),
       jnp.zeros((1, MAX_LEN - L_TARGET), dtype=jnp.bool_)], axis=1)
  am = jnp.where(in_audio, 1, 0)
  am = jnp.where(p < T_PROMPT, pna_full.astype(jnp.int32), am)
  am = jnp.where((p < L_TARGET) & tmask_full, 1, am)
  am = jnp.where(in_tgt_win, (p < tstart + tlen).astype(jnp.int32), am)
  am_ref[...] = am.reshape(1, 1, MAX_LEN)

  pos_ref[...] = p.reshape(1, 1, MAX_LEN)


def _meta(prompt_ids, target_ids, audio_lens):
  shp = jax.ShapeDtypeStruct((B, 1, MAX_LEN), jnp.int32)
  am, lab, pos = pl.pallas_call(
      _meta_kernel,
      grid=(B,),
      in_specs=[
          pl.BlockSpec((1, 1, T_PROMPT), lambda b: (b, 0, 0)),
          pl.BlockSpec((1, 1, L_TARGET), lambda b: (b, 0, 0)),
          pl.BlockSpec((1, 1, 1), lambda b: (b, 0, 0)),
      ],
      out_specs=[
          pl.BlockSpec((1, 1, MAX_LEN), lambda b: (b, 0, 0)),
          pl.BlockSpec((1, 1, MAX_LEN), lambda b: (b, 0, 0)),
          pl.BlockSpec((1, 1, MAX_LEN), lambda b: (b, 0, 0)),
      ],
      out_shape=[shp, shp, shp],
  )(prompt_ids.astype(jnp.int32).reshape(B, 1, T_PROMPT),
    target_ids.astype(jnp.int32).reshape(B, 1, L_TARGET),
    audio_lens.astype(jnp.int32).reshape(B, 1, 1))
  return (am.reshape(B, MAX_LEN), lab.reshape(B, MAX_LEN),
          pos.reshape(B, MAX_LEN))


def _routing(prompt_ids, target_ids, audio_lens):
  pids = prompt_ids.astype(jnp.int32)
  tids = target_ids.astype(jnp.int32)
  alen = audio_lens.astype(jnp.int32)
  tp = jnp.arange(T_PROMPT, dtype=jnp.int32)
  audio_pos = jnp.min(
      jnp.where(pids == AUDIO_TOKEN_ID, tp[None, :], T_PROMPT), axis=1)
  plen = jnp.sum(pids != PAD_TOKEN_ID, axis=1).astype(jnp.int32)
  tlen = jnp.sum(tids != PAD_TOKEN_ID, axis=1).astype(jnp.int32)
  cond = (alen < S_MAX) | (tlen < L_TARGET)
  rowbase = jnp.arange(B, dtype=jnp.int32) * MAX_LEN

  s = jnp.arange(E_SLOTS, dtype=jnp.int32)[None, :]
  is_prompt = s < P_SLOTS
  pv = jnp.minimum(s, audio_pos[:, None] - 1)
  pv = jnp.where(cond[:, None], jnp.maximum(pv, 1), pv)
  tj = jnp.minimum(jnp.maximum(s - P_SLOTS, 0), tlen[:, None] - 1)
  eidx = jnp.where(is_prompt,
                   jnp.take_along_axis(pids, jnp.minimum(pv, T_PROMPT - 1),
                                       axis=1),
                   jnp.take_along_axis(tids, tj, axis=1))
  edst = jnp.where(is_prompt,
                   rowbase[:, None] + pv,
                   rowbase[:, None] + alen[:, None] + plen[:, None] + tj)

  sa = jnp.arange(A_SLOTS, dtype=jnp.int32)[None, :]
  va = jnp.minimum(sa, alen[:, None] - 1)
  aidx = (jnp.arange(B, dtype=jnp.int32) * S_MAX)[:, None] + va
  adst = rowbase[:, None] + audio_pos[:, None] + va

  sz = jnp.arange(Z_SLOTS, dtype=jnp.int32)[None, :]
  gstart = audio_pos[:, None] + alen[:, None]
  glen = plen[:, None] - audio_pos[:, None]
  tstart = alen[:, None] + plen[:, None] + tlen[:, None]
  tail = MAX_LEN - tstart
  zrow = jnp.where(
      sz < glen, gstart + sz,
      jnp.where(sz < glen + tail, tstart + (sz - glen),
                jnp.where(cond[:, None], 0, gstart)))
  zdst = rowbase[:, None] + zrow

  def shape(a, ch):
    return a.reshape(32, -1, ch)

  return (shape(eidx, E_CH), shape(edst, E_CH), shape(aidx, A_CH),
          shape(adst, A_CH), shape(zdst, Z_CH))


def kernel(proj_embs, audio_lens, prompt_ids, target_ids, embed_table):
  et = lax.stop_gradient(embed_table).astype(jnp.float32)
  proj = proj_embs.astype(jnp.float32)
  eidx, edst, aidx, adst, zdst = _routing(prompt_ids, target_ids, audio_lens)
  zeros_src = jnp.zeros((Z_CH, D), jnp.float32)
  packed = _sc_pack(et, proj.reshape(B * S_MAX, D), zeros_src,
                    eidx, edst, aidx, adst, zdst)
  inputs_embeds = packed.reshape(B, MAX_LEN, D)
  attention_mask, labels, position_ids = _meta(prompt_ids, target_ids,
                                               audio_lens)
  audio_norm = _audio_norm(proj, audio_lens)
  text_norm = _text_norm(prompt_ids, et)
  return (inputs_embeds, attention_mask, labels, position_ids,
          audio_norm, text_norm)


# speedup vs baseline: 2.5660x; 2.5660x over previous
import functools

import jax
import jax.numpy as jnp
from jax import lax
from jax.experimental import pallas as pl
from jax.experimental.pallas import tpu as pltpu
from jax.experimental.pallas import tpu_sc as plsc

B, S_MAX, T_PROMPT, L_TARGET, D, V = 8, 512, 256, 256, 2048, 32000
AUDIO_TOKEN_ID = 5
PAD_TOKEN_ID = 0
MAX_LEN = S_MAX + T_PROMPT + L_TARGET

P_SLOTS = 32
E_SLOTS = 384
A_SLOTS = 512
Z_SLOTS = 640
E_CH, A_CH, Z_CH = 32, 32, 16
E_CHUNKS = B * E_SLOTS // (32 * E_CH)
A_CHUNKS = B * A_SLOTS // (32 * A_CH)
Z_CHUNKS = B * Z_SLOTS // (32 * Z_CH)


def _sc_pack(embed_table, proj_flat, zeros_src, eidx, edst, aidx, adst, zdst):
  info = plsc.get_sparse_core_info()
  NC, NS = info.num_cores, info.num_subcores
  mesh = plsc.VectorSubcoreMesh(core_axis_name="c", subcore_axis_name="s")

  @functools.partial(
      pl.kernel, mesh=mesh,
      out_type=jax.ShapeDtypeStruct((B * MAX_LEN, D), jnp.float32),
      scratch_types=[
          pltpu.VMEM((E_CHUNKS, E_CH), jnp.int32),
          pltpu.VMEM((E_CHUNKS, E_CH), jnp.int32),
          pltpu.VMEM((A_CHUNKS, A_CH), jnp.int32),
          pltpu.VMEM((A_CHUNKS, A_CH), jnp.int32),
          pltpu.VMEM((Z_CHUNKS, Z_CH), jnp.int32),
          pltpu.VMEM((E_CH, D), jnp.float32),
          pltpu.VMEM((Z_CH, D), jnp.float32),
          pltpu.SemaphoreType.DMA,
      ],
  )
  def k(table_hbm, proj_hbm, zsrc_hbm, eidx_hbm, edst_hbm, aidx_hbm,
        adst_hbm, zdst_hbm, out_hbm, eidx_v, edst_v, aidx_v, adst_v,
        zdst_v, rows_v, zbuf_v, sem):
    wid = lax.axis_index("s") * NC + lax.axis_index("c")
    pltpu.sync_copy(eidx_hbm.at[wid], eidx_v)
    pltpu.sync_copy(edst_hbm.at[wid], edst_v)
    pltpu.sync_copy(aidx_hbm.at[wid], aidx_v)
    pltpu.sync_copy(adst_hbm.at[wid], adst_v)
    pltpu.sync_copy(zdst_hbm.at[wid], zdst_v)
    pltpu.sync_copy(zsrc_hbm, zbuf_v)
    for j in range(E_CHUNKS):
      pltpu.async_copy(table_hbm.at[eidx_v.at[j]], rows_v, sem).wait()
      pltpu.async_copy(rows_v, out_hbm.at[edst_v.at[j]], sem).wait()
    for j in range(A_CHUNKS):
      pltpu.async_copy(proj_hbm.at[aidx_v.at[j]], rows_v, sem).wait()
      pltpu.async_copy(rows_v, out_hbm.at[adst_v.at[j]], sem).wait()
    for j in range(Z_CHUNKS):
      pltpu.async_copy(zbuf_v, out_hbm.at[zdst_v.at[j]], sem).wait()

  return k(embed_table, proj_flat, zeros_src, eidx, edst, aidx, adst, zdst)


TN_W = 16


def _text_norm_kernel(ids_ref, *refs):
  row_refs = refs[:TN_W]
  out_ref = refs[TN_W]
  acc_ref = refs[TN_W + 1]
  i = pl.program_id(0)
  n = pl.num_programs(0)

  @pl.when(i == 0)
  def _():
    acc_ref[0] = 0.0
    acc_ref[1] = 0.0

  tot = 0.0
  cnt = 0.0
  for k in range(TN_W):
    row = row_refs[k][0]
    valid = (ids_ref[i * TN_W + k] != PAD_TOKEN_ID).astype(jnp.float32)
    tot = tot + jnp.sqrt(jnp.sum(row * row)) * valid
    cnt = cnt + valid
  acc_ref[0] = acc_ref[0] + tot
  acc_ref[1] = acc_ref[1] + cnt

  @pl.when(i == n - 1)
  def _():
    out_ref[...] = (acc_ref[0] / jnp.maximum(acc_ref[1], 1.0)).reshape(1, 1)


def _text_norm(prompt_ids, embed_table):
  ids = prompt_ids.reshape(-1).astype(jnp.int32)

  def mk_spec(k):
    return pl.BlockSpec((1, 1, D), lambda i, ids, k=k: (ids[i * TN_W + k],
                                                        0, 0))

  grid_spec = pltpu.PrefetchScalarGridSpec(
      num_scalar_prefetch=1,
      grid=(B * T_PROMPT // TN_W,),
      in_specs=[mk_spec(k) for k in range(TN_W)],
      out_specs=pl.BlockSpec((1, 1), lambda i, ids: (0, 0)),
      scratch_shapes=[pltpu.SMEM((2,), jnp.float32)],
  )
  et3 = embed_table.reshape(V, 1, D)
  out = pl.pallas_call(
      _text_norm_kernel, grid_spec=grid_spec,
      out_shape=jax.ShapeDtypeStruct((1, 1), jnp.float32),
  )(ids, *([et3] * TN_W))
  return out[0, 0]


def _audio_norm_kernel(proj_ref, alen_ref, out_ref, acc_ref):
  b = pl.program_id(0)

  @pl.when(b == 0)
  def _():
    acc_ref[0] = 0.0
    acc_ref[1] = 0.0

  x = proj_ref[0]
  sumsq = jnp.sum(x * x, axis=-1)
  nrm = jnp.sqrt(sumsq)
  mask = (lax.broadcasted_iota(jnp.int32, (S_MAX,), 0)
          < alen_ref[0, 0, 0]).astype(jnp.float32)
  acc_ref[0] = acc_ref[0] + jnp.sum(nrm * mask)
  acc_ref[1] = acc_ref[1] + jnp.sum(mask)

  @pl.when(b == B - 1)
  def _():
    out_ref[...] = (acc_ref[0] / jnp.maximum(acc_ref[1], 1.0)).reshape(1, 1)


def _audio_norm(proj_embs, audio_lens):
  out = pl.pallas_call(
      _audio_norm_kernel,
      grid=(B,),
      in_specs=[
          pl.BlockSpec((1, S_MAX, D), lambda b: (b, 0, 0)),
          pl.BlockSpec((1, 1, 1), lambda b: (b, 0, 0)),
      ],
      out_specs=pl.BlockSpec((1, 1), lambda b: (0, 0)),
      scratch_shapes=[pltpu.SMEM((2,), jnp.float32)],
      out_shape=jax.ShapeDtypeStruct((1, 1), jnp.float32),
  )(proj_embs, audio_lens.astype(jnp.int32).reshape(B, 1, 1))
  return out[0, 0]


def _meta_kernel(pids_ref, tids_ref, alen_ref, am_ref, lab_ref, pos_ref):
  pids = pids_ref[0, 0].astype(jnp.int32)
  tids = tids_ref[0, 0].astype(jnp.int32)
  alen = alen_ref[0, 0, 0]
  p = lax.broadcasted_iota(jnp.int32, (1, MAX_LEN), 1)
  tp = lax.broadcasted_iota(jnp.int32, (1, T_PROMPT), 1)
  audio_pos = jnp.min(jnp.where(pids == AUDIO_TOKEN_ID, tp[0], T_PROMPT))
  plen = jnp.sum((pids != PAD_TOKEN_ID).astype(jnp.int32))
  tlen = jnp.sum((tids != PAD_TOKEN_ID).astype(jnp.int32))
  tstart = alen + plen

  oh = (p.reshape(MAX_LEN, 1) ==
        (tstart + lax.broadcasted_iota(jnp.int32, (1, L_TARGET), 1)
         .reshape(1, L_TARGET))).astype(jnp.float32)
  placed = jax.lax.dot_general(
      oh, tids.astype(jnp.float32).reshape(L_TARGET, 1),
      (((1,), (0,)), ((), ())),
      preferred_element_type=jnp.float32).reshape(1, MAX_LEN)
  in_tgt_valid = (p >= tstart) & (p < tstart + tlen)
  lab = jnp.where(in_tgt_valid, placed.astype(jnp.int32), -100)
  lab = jnp.where((p == 0) & (tlen < L_TARGET), 0, lab)
  lab_ref[...] = lab.reshape(1, 1, MAX_LEN)

  in_tgt_win = (p >= tstart) & (p < tstart + L_TARGET)
  in_audio = (p >= audio_pos) & (p < audio_pos + alen)
  pna = (pids != PAD_TOKEN_ID) & (pids != AUDIO_TOKEN_ID)
  pna_full = jnp.concatenate(
      [pna.reshape(1, T_PROMPT),
       jnp.zeros((1, MAX_LEN - T_PROMPT), dtype=jnp.bool_)], axis=1)
  tmask_full = jnp.concatenate(
      [(tids != PAD_TOKEN_ID).reshape(1, L_TARGET),
       jnp.zeros((1, MAX_LEN - L_TARGET), dtype=jnp.bool_)], axis=1)
  am = jnp.where(in_audio, 1, 0)
  am = jnp.where(p < T_PROMPT, pna_full.astype(jnp.int32), am)
  am = jnp.where((p < L_TARGET) & tmask_full, 1, am)
  am = jnp.where(in_tgt_win, (p < tstart + tlen).astype(jnp.int32), am)
  am_ref[...] = am.reshape(1, 1, MAX_LEN)

  pos_ref[...] = p.reshape(1, 1, MAX_LEN)


def _meta(prompt_ids, target_ids, audio_lens):
  shp = jax.ShapeDtypeStruct((B, 1, MAX_LEN), jnp.int32)
  am, lab, pos = pl.pallas_call(
      _meta_kernel,
      grid=(B,),
      in_specs=[
          pl.BlockSpec((1, 1, T_PROMPT), lambda b: (b, 0, 0)),
          pl.BlockSpec((1, 1, L_TARGET), lambda b: (b, 0, 0)),
          pl.BlockSpec((1, 1, 1), lambda b: (b, 0, 0)),
      ],
      out_specs=[
          pl.BlockSpec((1, 1, MAX_LEN), lambda b: (b, 0, 0)),
          pl.BlockSpec((1, 1, MAX_LEN), lambda b: (b, 0, 0)),
          pl.BlockSpec((1, 1, MAX_LEN), lambda b: (b, 0, 0)),
      ],
      out_shape=[shp, shp, shp],
  )(prompt_ids.astype(jnp.int32).reshape(B, 1, T_PROMPT),
    target_ids.astype(jnp.int32).reshape(B, 1, L_TARGET),
    audio_lens.astype(jnp.int32).reshape(B, 1, 1))
  return (am.reshape(B, MAX_LEN), lab.reshape(B, MAX_LEN),
          pos.reshape(B, MAX_LEN))


def _routing(prompt_ids, target_ids, audio_lens):
  pids = prompt_ids.astype(jnp.int32)
  tids = target_ids.astype(jnp.int32)
  alen = audio_lens.astype(jnp.int32)
  tp = jnp.arange(T_PROMPT, dtype=jnp.int32)
  audio_pos = jnp.min(
      jnp.where(pids == AUDIO_TOKEN_ID, tp[None, :], T_PROMPT), axis=1)
  plen = jnp.sum(pids != PAD_TOKEN_ID, axis=1).astype(jnp.int32)
  tlen = jnp.sum(tids != PAD_TOKEN_ID, axis=1).astype(jnp.int32)
  cond = (alen < S_MAX) | (tlen < L_TARGET)
  rowbase = jnp.arange(B, dtype=jnp.int32) * MAX_LEN

  s = jnp.arange(E_SLOTS, dtype=jnp.int32)[None, :]
  is_prompt = s < P_SLOTS
  pv = jnp.minimum(s, audio_pos[:, None] - 1)
  pv = jnp.where(cond[:, None], jnp.maximum(pv, 1), pv)
  tj = jnp.minimum(jnp.maximum(s - P_SLOTS, 0), tlen[:, None] - 1)
  eidx = jnp.where(is_prompt,
                   jnp.take_along_axis(pids, jnp.minimum(pv, T_PROMPT - 1),
                                       axis=1),
                   jnp.take_along_axis(tids, tj, axis=1))
  edst = jnp.where(is_prompt,
                   rowbase[:, None] + pv,
                   rowbase[:, None] + alen[:, None] + plen[:, None] + tj)

  sa = jnp.arange(A_SLOTS, dtype=jnp.int32)[None, :]
  va = jnp.minimum(sa, alen[:, None] - 1)
  aidx = (jnp.arange(B, dtype=jnp.int32) * S_MAX)[:, None] + va
  adst = rowbase[:, None] + audio_pos[:, None] + va

  sz = jnp.arange(Z_SLOTS, dtype=jnp.int32)[None, :]
  gstart = audio_pos[:, None] + alen[:, None]
  glen = plen[:, None] - audio_pos[:, None]
  tstart = alen[:, None] + plen[:, None] + tlen[:, None]
  tail = MAX_LEN - tstart
  zrow = jnp.where(
      sz < glen, gstart + sz,
      jnp.where(sz < glen + tail, tstart + (sz - glen),
                jnp.where(cond[:, None], 0, gstart)))
  zdst = rowbase[:, None] + zrow

  def shape(a, ch):
    return a.reshape(32, -1, ch)

  return (shape(eidx, E_CH), shape(edst, E_CH), shape(aidx, A_CH),
          shape(adst, A_CH), shape(zdst, Z_CH))


def kernel(proj_embs, audio_lens, prompt_ids, target_ids, embed_table):
  et = lax.stop_gradient(embed_table).astype(jnp.float32)
  proj = proj_embs.astype(jnp.float32)
  eidx, edst, aidx, adst, zdst = _routing(prompt_ids, target_ids, audio_lens)
  zeros_src = jnp.zeros((Z_CH, D), jnp.float32)
  packed = _sc_pack(et, proj.reshape(B * S_MAX, D), zeros_src,
                    eidx, edst, aidx, adst, zdst)
  inputs_embeds = packed.reshape(B, MAX_LEN, D)
  attention_mask, labels, position_ids = _meta(prompt_ids, target_ids,
                                               audio_lens)
  audio_norm = _audio_norm(proj, audio_lens)
  text_norm = _text_norm(prompt_ids, et)
  return (inputs_embeds, attention_mask, labels, position_ids,
          audio_norm, text_norm)


# speedup vs baseline: 2.6532x; 1.0340x over previous
import functools

import jax
import jax.numpy as jnp
from jax import lax
from jax.experimental import pallas as pl
from jax.experimental.pallas import tpu as pltpu
from jax.experimental.pallas import tpu_sc as plsc

B, S_MAX, T_PROMPT, L_TARGET, D, V = 8, 512, 256, 256, 2048, 32000
AUDIO_TOKEN_ID = 5
PAD_TOKEN_ID = 0
MAX_LEN = S_MAX + T_PROMPT + L_TARGET

P_SLOTS = 32
E_SLOTS = 384
A_SLOTS = 512
Z_SLOTS = 640
E_CH, A_CH, Z_CH = 16, 16, 16
E_CHUNKS = B * E_SLOTS // (32 * E_CH)
A_CHUNKS = B * A_SLOTS // (32 * A_CH)
Z_CHUNKS = B * Z_SLOTS // (32 * Z_CH)


def _sc_pack(embed_table, proj_flat, zeros_src, eidx, edst, aidx, adst, zdst):
  info = plsc.get_sparse_core_info()
  NC, NS = info.num_cores, info.num_subcores
  mesh = plsc.VectorSubcoreMesh(core_axis_name="c", subcore_axis_name="s")

  @functools.partial(
      pl.kernel, mesh=mesh,
      out_type=jax.ShapeDtypeStruct((B * MAX_LEN, D), jnp.float32),
      scratch_types=[
          pltpu.VMEM((E_CHUNKS, E_CH), jnp.int32),
          pltpu.VMEM((E_CHUNKS, E_CH), jnp.int32),
          pltpu.VMEM((A_CHUNKS, A_CH), jnp.int32),
          pltpu.VMEM((A_CHUNKS, A_CH), jnp.int32),
          pltpu.VMEM((Z_CHUNKS, Z_CH), jnp.int32),
          pltpu.VMEM((E_CH, D), jnp.float32),
          pltpu.VMEM((E_CH, D), jnp.float32),
          pltpu.VMEM((Z_CH, D), jnp.float32),
          pltpu.SemaphoreType.DMA,
          pltpu.SemaphoreType.DMA,
          pltpu.SemaphoreType.DMA,
          pltpu.SemaphoreType.DMA,
          pltpu.SemaphoreType.DMA,
      ],
  )
  def k(table_hbm, proj_hbm, zsrc_hbm, eidx_hbm, edst_hbm, aidx_hbm,
        adst_hbm, zdst_hbm, out_hbm, eidx_v, edst_v, aidx_v, adst_v,
        zdst_v, rows0_v, rows1_v, zbuf_v, g0, g1, s0, s1, zs):
    wid = lax.axis_index("s") * NC + lax.axis_index("c")
    pltpu.sync_copy(eidx_hbm.at[wid], eidx_v)
    pltpu.sync_copy(edst_hbm.at[wid], edst_v)
    pltpu.sync_copy(aidx_hbm.at[wid], aidx_v)
    pltpu.sync_copy(adst_hbm.at[wid], adst_v)
    pltpu.sync_copy(zdst_hbm.at[wid], zdst_v)
    pltpu.sync_copy(zsrc_hbm, zbuf_v)

    zcopies = [pltpu.make_async_copy(zbuf_v, out_hbm.at[zdst_v.at[j]], zs)
               for j in range(Z_CHUNKS)]
    for c in zcopies:
      c.start()

    seq = ([(table_hbm, eidx_v, edst_v, j) for j in range(E_CHUNKS)]
           + [(proj_hbm, aidx_v, adst_v, j) for j in range(A_CHUNKS)])
    n = len(seq)
    bufs, gsems, ssems = [rows0_v, rows1_v], [g0, g1], [s0, s1]
    gc, sc = [None] * n, [None] * n

    def start_gather(k):
      src, iv, _, j = seq[k]
      b = k & 1
      gc[k] = pltpu.make_async_copy(src.at[iv.at[j]], bufs[b], gsems[b])
      gc[k].start()

    def start_scatter(k):
      _, _, dv, j = seq[k]
      b = k & 1
      sc[k] = pltpu.make_async_copy(bufs[b], out_hbm.at[dv.at[j]], ssems[b])
      sc[k].start()

    start_gather(0)
    for k in range(1, n):
      if k >= 2:
        sc[k - 2].wait()
      start_gather(k)
      gc[k - 1].wait()
      start_scatter(k - 1)
    gc[n - 1].wait()
    start_scatter(n - 1)
    sc[n - 2].wait()
    sc[n - 1].wait()
    for c in zcopies:
      c.wait()

  return k(embed_table, proj_flat, zeros_src, eidx, edst, aidx, adst, zdst)


TN_W = 32


def _text_norm_kernel(ids_ref, *refs):
  row_refs = refs[:TN_W]
  out_ref = refs[TN_W]
  acc_ref = refs[TN_W + 1]
  i = pl.program_id(0)
  n = pl.num_programs(0)

  @pl.when(i == 0)
  def _():
    acc_ref[0] = 0.0
    acc_ref[1] = 0.0

  tot = 0.0
  cnt = 0.0
  for k in range(TN_W):
    row = row_refs[k][0]
    valid = (ids_ref[i * TN_W + k] != PAD_TOKEN_ID).astype(jnp.float32)
    tot = tot + jnp.sqrt(jnp.sum(row * row)) * valid
    cnt = cnt + valid
  acc_ref[0] = acc_ref[0] + tot
  acc_ref[1] = acc_ref[1] + cnt

  @pl.when(i == n - 1)
  def _():
    out_ref[...] = (acc_ref[0] / jnp.maximum(acc_ref[1], 1.0)).reshape(1, 1)


def _text_norm(prompt_ids, embed_table):
  ids = prompt_ids.reshape(-1).astype(jnp.int32)

  def mk_spec(k):
    return pl.BlockSpec((1, 1, D), lambda i, ids, k=k: (ids[i * TN_W + k],
                                                        0, 0))

  grid_spec = pltpu.PrefetchScalarGridSpec(
      num_scalar_prefetch=1,
      grid=(B * T_PROMPT // TN_W,),
      in_specs=[mk_spec(k) for k in range(TN_W)],
      out_specs=pl.BlockSpec((1, 1), lambda i, ids: (0, 0)),
      scratch_shapes=[pltpu.SMEM((2,), jnp.float32)],
  )
  et3 = embed_table.reshape(V, 1, D)
  out = pl.pallas_call(
      _text_norm_kernel, grid_spec=grid_spec,
      out_shape=jax.ShapeDtypeStruct((1, 1), jnp.float32),
  )(ids, *([et3] * TN_W))
  return out[0, 0]


def _audio_norm_kernel(proj_ref, alen_ref, out_ref, acc_ref):
  b = pl.program_id(0)

  @pl.when(b == 0)
  def _():
    acc_ref[0] = 0.0
    acc_ref[1] = 0.0

  x = proj_ref[0]
  sumsq = jnp.sum(x * x, axis=-1)
  nrm = jnp.sqrt(sumsq)
  mask = (lax.broadcasted_iota(jnp.int32, (S_MAX,), 0)
          < alen_ref[0, 0, 0]).astype(jnp.float32)
  acc_ref[0] = acc_ref[0] + jnp.sum(nrm * mask)
  acc_ref[1] = acc_ref[1] + jnp.sum(mask)

  @pl.when(b == B - 1)
  def _():
    out_ref[...] = (acc_ref[0] / jnp.maximum(acc_ref[1], 1.0)).reshape(1, 1)


def _audio_norm(proj_embs, audio_lens):
  out = pl.pallas_call(
      _audio_norm_kernel,
      grid=(B,),
      in_specs=[
          pl.BlockSpec((1, S_MAX, D), lambda b: (b, 0, 0)),
          pl.BlockSpec((1, 1, 1), lambda b: (b, 0, 0)),
      ],
      out_specs=pl.BlockSpec((1, 1), lambda b: (0, 0)),
      scratch_shapes=[pltpu.SMEM((2,), jnp.float32)],
      out_shape=jax.ShapeDtypeStruct((1, 1), jnp.float32),
  )(proj_embs, audio_lens.astype(jnp.int32).reshape(B, 1, 1))
  return out[0, 0]


def _meta_kernel(pids_ref, tids_ref, alen_ref, am_ref, lab_ref, pos_ref):
  pids = pids_ref[0, 0].astype(jnp.int32)
  tids = tids_ref[0, 0].astype(jnp.int32)
  alen = alen_ref[0, 0, 0]
  p = lax.broadcasted_iota(jnp.int32, (1, MAX_LEN), 1)
  tp = lax.broadcasted_iota(jnp.int32, (1, T_PROMPT), 1)
  audio_pos = jnp.min(jnp.where(pids == AUDIO_TOKEN_ID, tp[0], T_PROMPT))
  plen = jnp.sum((pids != PAD_TOKEN_ID).astype(jnp.int32))
  tlen = jnp.sum((tids != PAD_TOKEN_ID).astype(jnp.int32))
  tstart = alen + plen

  oh = (p.reshape(MAX_LEN, 1) ==
        (tstart + lax.broadcasted_iota(jnp.int32, (1, L_TARGET), 1)
         .reshape(1, L_TARGET))).astype(jnp.float32)
  placed = jax.lax.dot_general(
      oh, tids.astype(jnp.float32).reshape(L_TARGET, 1),
      (((1,), (0,)), ((), ())),
      preferred_element_type=jnp.float32).reshape(1, MAX_LEN)
  in_tgt_valid = (p >= tstart) & (p < tstart + tlen)
  lab = jnp.where(in_tgt_valid, placed.astype(jnp.int32), -100)
  lab = jnp.where((p == 0) & (tlen < L_TARGET), 0, lab)
  lab_ref[...] = lab.reshape(1, 1, MAX_LEN)

  in_tgt_win = (p >= tstart) & (p < tstart + L_TARGET)
  in_audio = (p >= audio_pos) & (p < audio_pos + alen)
  pna = (pids != PAD_TOKEN_ID) & (pids != AUDIO_TOKEN_ID)
  pna_full = jnp.concatenate(
      [pna.reshape(1, T_PROMPT),
       jnp.zeros((1, MAX_LEN - T_PROMPT), dtype=jnp.bool_)], axis=1)
  tmask_full = jnp.concatenate(
      [(tids != PAD_TOKEN_ID).reshape(1, L_TARGET),
       jnp.zeros((1, MAX_LEN - L_TARGET), dtype=jnp.bool_)], axis=1)
  am = jnp.where(in_audio, 1, 0)
  am = jnp.where(p < T_PROMPT, pna_full.astype(jnp.int32), am)
  am = jnp.where((p < L_TARGET) & tmask_full, 1, am)
  am = jnp.where(in_tgt_win, (p < tstart + tlen).astype(jnp.int32), am)
  am_ref[...] = am.reshape(1, 1, MAX_LEN)

  pos_ref[...] = p.reshape(1, 1, MAX_LEN)


def _meta(prompt_ids, target_ids, audio_lens):
  shp = jax.ShapeDtypeStruct((B, 1, MAX_LEN), jnp.int32)
  am, lab, pos = pl.pallas_call(
      _meta_kernel,
      grid=(B,),
      in_specs=[
          pl.BlockSpec((1, 1, T_PROMPT), lambda b: (b, 0, 0)),
          pl.BlockSpec((1, 1, L_TARGET), lambda b: (b, 0, 0)),
          pl.BlockSpec((1, 1, 1), lambda b: (b, 0, 0)),
      ],
      out_specs=[
          pl.BlockSpec((1, 1, MAX_LEN), lambda b: (b, 0, 0)),
          pl.BlockSpec((1, 1, MAX_LEN), lambda b: (b, 0, 0)),
          pl.BlockSpec((1, 1, MAX_LEN), lambda b: (b, 0, 0)),
      ],
      out_shape=[shp, shp, shp],
  )(prompt_ids.astype(jnp.int32).reshape(B, 1, T_PROMPT),
    target_ids.astype(jnp.int32).reshape(B, 1, L_TARGET),
    audio_lens.astype(jnp.int32).reshape(B, 1, 1))
  return (am.reshape(B, MAX_LEN), lab.reshape(B, MAX_LEN),
          pos.reshape(B, MAX_LEN))


def _routing(prompt_ids, target_ids, audio_lens):
  pids = prompt_ids.astype(jnp.int32)
  tids = target_ids.astype(jnp.int32)
  alen = audio_lens.astype(jnp.int32)
  tp = jnp.arange(T_PROMPT, dtype=jnp.int32)
  audio_pos = jnp.min(
      jnp.where(pids == AUDIO_TOKEN_ID, tp[None, :], T_PROMPT), axis=1)
  plen = jnp.sum(pids != PAD_TOKEN_ID, axis=1).astype(jnp.int32)
  tlen = jnp.sum(tids != PAD_TOKEN_ID, axis=1).astype(jnp.int32)
  cond = (alen < S_MAX) | (tlen < L_TARGET)
  rowbase = jnp.arange(B, dtype=jnp.int32) * MAX_LEN

  s = jnp.arange(E_SLOTS, dtype=jnp.int32)[None, :]
  is_prompt = s < P_SLOTS
  pv = jnp.minimum(s, audio_pos[:, None] - 1)
  pv = jnp.where(cond[:, None], jnp.maximum(pv, 1), pv)
  tj = jnp.minimum(jnp.maximum(s - P_SLOTS, 0), tlen[:, None] - 1)
  eidx = jnp.where(is_prompt,
                   jnp.take_along_axis(pids, jnp.minimum(pv, T_PROMPT - 1),
                                       axis=1),
                   jnp.take_along_axis(tids, tj, axis=1))
  edst = jnp.where(is_prompt,
                   rowbase[:, None] + pv,
                   rowbase[:, None] + alen[:, None] + plen[:, None] + tj)

  sa = jnp.arange(A_SLOTS, dtype=jnp.int32)[None, :]
  va = jnp.minimum(sa, alen[:, None] - 1)
  aidx = (jnp.arange(B, dtype=jnp.int32) * S_MAX)[:, None] + va
  adst = rowbase[:, None] + audio_pos[:, None] + va

  sz = jnp.arange(Z_SLOTS, dtype=jnp.int32)[None, :]
  gstart = audio_pos[:, None] + alen[:, None]
  glen = plen[:, None] - audio_pos[:, None]
  tstart = alen[:, None] + plen[:, None] + tlen[:, None]
  tail = MAX_LEN - tstart
  zrow = jnp.where(
      sz < glen, gstart + sz,
      jnp.where(sz < glen + tail, tstart + (sz - glen),
                jnp.where(cond[:, None], 0, gstart)))
  zdst = rowbase[:, None] + zrow

  def shape(a, ch):
    return a.reshape(32, -1, ch)

  return (shape(eidx, E_CH), shape(edst, E_CH), shape(aidx, A_CH),
          shape(adst, A_CH), shape(zdst, Z_CH))


def kernel(proj_embs, audio_lens, prompt_ids, target_ids, embed_table):
  et = lax.stop_gradient(embed_table).astype(jnp.float32)
  proj = proj_embs.astype(jnp.float32)
  eidx, edst, aidx, adst, zdst = _routing(prompt_ids, target_ids, audio_lens)
  zeros_src = jnp.zeros((Z_CH, D), jnp.float32)
  packed = _sc_pack(et, proj.reshape(B * S_MAX, D), zeros_src,
                    eidx, edst, aidx, adst, zdst)
  inputs_embeds = packed.reshape(B, MAX_LEN, D)
  attention_mask, labels, position_ids = _meta(prompt_ids, target_ids,
                                               audio_lens)
  audio_norm = _audio_norm(proj, audio_lens)
  text_norm = _text_norm(prompt_ids, et)
  return (inputs_embeds, attention_mask, labels, position_ids,
          audio_norm, text_norm)
